# baseline (speedup 1.0000x reference)
import functools

import jax
import jax.numpy as jnp
from jax.experimental import pallas as pl
from jax.experimental.pallas import tpu as pltpu

_BLK = 256
_NBUF = 4


def _gcn_kernel(inter_ref, adj1_hbm, adj2_hbm, e1_ref, e2_ref,
                o1_ref, o2_ref, buf1, buf2, sem1, sem2, *, blk, nbuf, half):
    n = e1_ref.shape[0]
    nchunks = n // blk

    def _copy(j, slot, hbm, buf, sem):
        return pltpu.make_async_copy(
            hbm.at[pl.ds(j * blk, blk), :], buf.at[slot], sem.at[slot])

    for s in range(nbuf - 1):
        _copy(s, s, adj1_hbm, buf1, sem1).start()
        _copy(s, s, adj2_hbm, buf2, sem2).start()

    e1 = e1_ref[...]
    e2 = e2_ref[...]
    w = inter_ref[0]

    def body(j, carry):
        ahead = j + nbuf - 1
        fill = jax.lax.rem(ahead, nbuf)

        @pl.when(ahead < nchunks)
        def _():
            _copy(ahead, fill, adj1_hbm, buf1, sem1).start()
            _copy(ahead, fill, adj2_hbm, buf2, sem2).start()

        slot = jax.lax.rem(j, nbuf)
        _copy(j, slot, adj1_hbm, buf1, sem1).wait()
        _copy(j, slot, adj2_hbm, buf2, sem2).wait()
        y1 = jnp.dot(buf1[slot], e1, preferred_element_type=jnp.float32)
        y2 = jnp.dot(buf2[slot], e2, preferred_element_type=jnp.float32)
        t1 = 4.0 * jnp.maximum(y1, 0.0)
        t2 = 4.0 * jnp.maximum(y2, 0.0)
        o1_ref[pl.ds(j * blk, blk), :] = t1

        @pl.when(j < half)
        def _():
            o2_ref[pl.ds(j * blk, blk), :] = t2

        @pl.when(j >= half)
        def _():
            o2_ref[pl.ds(j * blk, blk), :] = w * t1 + (1.0 - w) * t2

        return carry

    jax.lax.fori_loop(0, nchunks, body, 0)


def kernel(adj1, adj2, dEmbed, mEmbed, pEmbed, inter):
    e1 = jnp.concatenate([dEmbed, mEmbed], axis=0)
    e2 = jnp.concatenate([pEmbed, mEmbed], axis=0)
    n = adj1.shape[0]
    d = dEmbed.shape[0]
    p = pEmbed.shape[0]
    f = dEmbed.shape[1]
    blk = _BLK
    nbuf = _NBUF
    half = d // blk

    o1, o2 = pl.pallas_call(
        functools.partial(_gcn_kernel, blk=blk, nbuf=nbuf, half=half),
        grid=(1,),
        in_specs=[
            pl.BlockSpec(memory_space=pltpu.SMEM),
            pl.BlockSpec(memory_space=pl.ANY),
            pl.BlockSpec(memory_space=pl.ANY),
            pl.BlockSpec((n, f), lambda i: (0, 0)),
            pl.BlockSpec((n, f), lambda i: (0, 0)),
        ],
        out_specs=[
            pl.BlockSpec((n, f), lambda i: (0, 0)),
            pl.BlockSpec((n, f), lambda i: (0, 0)),
        ],
        out_shape=[
            jax.ShapeDtypeStruct((n, f), jnp.float32),
            jax.ShapeDtypeStruct((n, f), jnp.float32),
        ],
        scratch_shapes=[
            pltpu.VMEM((nbuf, blk, n), jnp.float32),
            pltpu.VMEM((nbuf, blk, n), jnp.float32),
            pltpu.SemaphoreType.DMA((nbuf,)),
            pltpu.SemaphoreType.DMA((nbuf,)),
        ],
    )(inter, adj1, adj2, e1, e2)
    return (o2[p:], o1[:d], o2[:p])


# speedup vs baseline: 1.0031x; 1.0031x over previous
import functools

import jax
import jax.numpy as jnp
from jax.experimental import pallas as pl
from jax.experimental.pallas import tpu as pltpu

_BLK = 256
_NBUF = 4


def _gcn_kernel(inter_ref, adj1_hbm, adj2_hbm, e1_ref, e2_ref,
                o1_ref, o2_ref, buf1, buf2, sem1, sem2, *, blk, nbuf, half):
    n = e1_ref.shape[0]
    nchunks = n // blk

    hb = blk // 2

    def _copies(j, slot, hbm, buf, sem):
        lo = pltpu.make_async_copy(
            hbm.at[pl.ds(j * blk, hb), :], buf.at[slot, pl.ds(0, hb), :],
            sem.at[slot, 0])
        hi = pltpu.make_async_copy(
            hbm.at[pl.ds(j * blk + hb, hb), :], buf.at[slot, pl.ds(hb, hb), :],
            sem.at[slot, 1])
        return lo, hi

    def _start(j, slot, hbm, buf, sem):
        lo, hi = _copies(j, slot, hbm, buf, sem)
        lo.start()
        hi.start()

    def _wait(j, slot, hbm, buf, sem):
        lo, hi = _copies(j, slot, hbm, buf, sem)
        lo.wait()
        hi.wait()

    for s in range(nbuf - 1):
        _start(s, s, adj1_hbm, buf1, sem1)
        _start(s, s, adj2_hbm, buf2, sem2)

    e1 = e1_ref[...]
    e2 = e2_ref[...]
    w = inter_ref[0]

    def body(j, carry):
        ahead = j + nbuf - 1
        fill = jax.lax.rem(ahead, nbuf)

        @pl.when(ahead < nchunks)
        def _():
            _start(ahead, fill, adj1_hbm, buf1, sem1)
            _start(ahead, fill, adj2_hbm, buf2, sem2)

        slot = jax.lax.rem(j, nbuf)
        _wait(j, slot, adj1_hbm, buf1, sem1)
        _wait(j, slot, adj2_hbm, buf2, sem2)
        y1 = jnp.dot(buf1[slot], e1, preferred_element_type=jnp.float32)
        y2 = jnp.dot(buf2[slot], e2, preferred_element_type=jnp.float32)
        t1 = 4.0 * jnp.maximum(y1, 0.0)
        t2 = 4.0 * jnp.maximum(y2, 0.0)
        o1_ref[pl.ds(j * blk, blk), :] = t1

        @pl.when(j < half)
        def _():
            o2_ref[pl.ds(j * blk, blk), :] = t2

        @pl.when(j >= half)
        def _():
            o2_ref[pl.ds(j * blk, blk), :] = w * t1 + (1.0 - w) * t2

        return carry

    jax.lax.fori_loop(0, nchunks, body, 0)


def kernel(adj1, adj2, dEmbed, mEmbed, pEmbed, inter):
    e1 = jnp.concatenate([dEmbed, mEmbed], axis=0)
    e2 = jnp.concatenate([pEmbed, mEmbed], axis=0)
    n = adj1.shape[0]
    d = dEmbed.shape[0]
    p = pEmbed.shape[0]
    f = dEmbed.shape[1]
    blk = _BLK
    nbuf = _NBUF
    half = d // blk

    o1, o2 = pl.pallas_call(
        functools.partial(_gcn_kernel, blk=blk, nbuf=nbuf, half=half),
        grid=(1,),
        in_specs=[
            pl.BlockSpec(memory_space=pltpu.SMEM),
            pl.BlockSpec(memory_space=pl.ANY),
            pl.BlockSpec(memory_space=pl.ANY),
            pl.BlockSpec((n, f), lambda i: (0, 0)),
            pl.BlockSpec((n, f), lambda i: (0, 0)),
        ],
        out_specs=[
            pl.BlockSpec((n, f), lambda i: (0, 0)),
            pl.BlockSpec((n, f), lambda i: (0, 0)),
        ],
        out_shape=[
            jax.ShapeDtypeStruct((n, f), jnp.float32),
            jax.ShapeDtypeStruct((n, f), jnp.float32),
        ],
        scratch_shapes=[
            pltpu.VMEM((nbuf, blk, n), jnp.float32),
            pltpu.VMEM((nbuf, blk, n), jnp.float32),
            pltpu.SemaphoreType.DMA((nbuf, 2)),
            pltpu.SemaphoreType.DMA((nbuf, 2)),
        ],
    )(inter, adj1, adj2, e1, e2)
    return (o2[p:], o1[:d], o2[:p])
